# initial kernel scaffold (unmeasured)
import jax
import jax.numpy as jnp
from jax import lax
from jax.experimental import pallas as pl
from jax.experimental.pallas import tpu as pltpu

N_DEV = 4
H_PER = 8
BLK = 64
SCALE = 0.08838834764831843


def kernel(x, Wq, K_ext, V_ext, Wo):
    B, Sq, Dm = x.shape
    _, Skv, Hq, Dh = K_ext.shape
    Dq = Wq.shape[1]

    def body(x_ref, wq_ref, k_ref, v_ref, wo_ref, out_ref,
             kbuf, vbuf, ctx_ref, pbuf, rbuf1, sbuf2, rbuf2,
             kv_sems, send_sems, recv_sems):
        my = lax.axis_index("i")
        head_base = my * H_PER

        def kv_copies(h, slot):
            kc = pltpu.make_async_copy(
                k_ref.at[0, :, head_base + h, :], kbuf.at[slot],
                kv_sems.at[slot, 0])
            vc = pltpu.make_async_copy(
                v_ref.at[0, :, head_base + h, :], vbuf.at[slot],
                kv_sems.at[slot, 1])
            return kc, vc

        pending = kv_copies(0, 0)
        pending[0].start()
        pending[1].start()

        xb = x_ref[0].astype(jnp.bfloat16)
        wqb = wq_ref[...].astype(jnp.bfloat16)
        q = jnp.dot(xb, wqb, preferred_element_type=jnp.float32) * SCALE
        qb16 = q.astype(jnp.bfloat16)

        qblk = lax.broadcasted_iota(jnp.int32, (Sq, Skv), 0) // BLK
        kblk = lax.broadcasted_iota(jnp.int32, (Sq, Skv), 1) // BLK
        mask = (qblk == kblk) | (kblk == 0) | ((qblk + kblk) % 3 == 0)

        for h in range(H_PER):
            slot = h % 2
            pending[0].wait()
            pending[1].wait()
            if h + 1 < H_PER:
                pending = kv_copies(h + 1, (h + 1) % 2)
                pending[0].start()
                pending[1].start()
            qh = qb16[:, h * Dh:(h + 1) * Dh]
            kh = kbuf[slot].astype(jnp.bfloat16)
            scores = lax.dot_general(
                qh, kh, (((1,), (1,)), ((), ())),
                preferred_element_type=jnp.float32)
            scores = jnp.where(mask, scores, -1e9)
            m = jnp.max(scores, axis=1, keepdims=True)
            w = jnp.exp(scores - m)
            s = jnp.sum(w, axis=1, keepdims=True)
            vh = vbuf[slot].astype(jnp.bfloat16)
            ctx_h = jnp.dot(w.astype(jnp.bfloat16), vh,
                            preferred_element_type=jnp.float32) / s
            ctx_ref[:, h * Dh:(h + 1) * Dh] = ctx_h.astype(jnp.bfloat16)

        wob = wo_ref[...].astype(jnp.bfloat16)
        partial = jnp.dot(ctx_ref[...], wob,
                          preferred_element_type=jnp.float32)
        pbuf[...] = partial.astype(jnp.bfloat16)

        barrier = pltpu.get_barrier_semaphore()
        p1 = jnp.bitwise_xor(my, 1)
        p2 = 3 - my
        for nbr in (p1, p2):
            pl.semaphore_signal(barrier, inc=1, device_id=(nbr,),
                                device_id_type=pl.DeviceIdType.MESH)
        pl.semaphore_wait(barrier, 2)

        ex1 = pltpu.make_async_remote_copy(
            src_ref=pbuf, dst_ref=rbuf1,
            send_sem=send_sems.at[0], recv_sem=recv_sems.at[0],
            device_id=(p1,), device_id_type=pl.DeviceIdType.MESH)
        ex1.start()
        ex1.wait()
        sum1 = partial + rbuf1[...].astype(jnp.float32)
        sbuf2[...] = sum1.astype(jnp.bfloat16)

        ex2 = pltpu.make_async_remote_copy(
            src_ref=sbuf2, dst_ref=rbuf2,
            send_sem=send_sems.at[1], recv_sem=recv_sems.at[1],
            device_id=(p2,), device_id_type=pl.DeviceIdType.MESH)
        ex2.start()
        ex2.wait()
        out_ref[0] = sum1 + rbuf2[...].astype(jnp.float32)

    return pl.pallas_call(
        body,
        out_shape=jax.ShapeDtypeStruct((B, Sq, Dm), jnp.float32),
        in_specs=[
            pl.BlockSpec(memory_space=pltpu.VMEM),
            pl.BlockSpec(memory_space=pltpu.VMEM),
            pl.BlockSpec(memory_space=pltpu.ANY),
            pl.BlockSpec(memory_space=pltpu.ANY),
            pl.BlockSpec(memory_space=pltpu.VMEM),
        ],
        out_specs=pl.BlockSpec(memory_space=pltpu.VMEM),
        scratch_shapes=[
            pltpu.VMEM((2, Skv, Dh), jnp.float32),
            pltpu.VMEM((2, Skv, Dh), jnp.float32),
            pltpu.VMEM((Sq, Dq), jnp.bfloat16),
            pltpu.VMEM((Sq, Dm), jnp.bfloat16),
            pltpu.VMEM((Sq, Dm), jnp.bfloat16),
            pltpu.VMEM((Sq, Dm), jnp.bfloat16),
            pltpu.VMEM((Sq, Dm), jnp.bfloat16),
            pltpu.SemaphoreType.DMA((2, 2)),
            pltpu.SemaphoreType.DMA((2,)),
            pltpu.SemaphoreType.DMA((2,)),
        ],
        compiler_params=pltpu.CompilerParams(collective_id=0),
    )(x, Wq, K_ext, V_ext, Wo)


# baseline (device time: 48671 ns/iter reference)
import jax
import jax.numpy as jnp
from jax import lax
from jax.experimental import pallas as pl
from jax.experimental.pallas import tpu as pltpu

N_DEV = 4
H_PER = 8
BLK = 64
SCALE = 0.08838834764831843


def kernel(x, Wq, K_ext, V_ext, Wo):
    B, Sq, Dm = x.shape
    _, Skv, Hq, Dh = K_ext.shape
    Dq = Wq.shape[1]

    def body(x_ref, wq_ref, k_ref, v_ref, wo_ref, out_ref,
             kbuf, vbuf, ctx_ref, pbuf, rbuf1, sbuf2, rbuf2,
             kv_sems, send_sems, recv_sems):
        my = lax.axis_index("i")
        head_base = my * H_PER

        def kv_copies(h, slot):
            kc = pltpu.make_async_copy(
                k_ref.at[0, :, head_base + h, :], kbuf.at[slot],
                kv_sems.at[slot, 0])
            vc = pltpu.make_async_copy(
                v_ref.at[0, :, head_base + h, :], vbuf.at[slot],
                kv_sems.at[slot, 1])
            return kc, vc

        pending = kv_copies(0, 0)
        pending[0].start()
        pending[1].start()

        xb = x_ref[0].astype(jnp.bfloat16)
        wqb = wq_ref[...].astype(jnp.bfloat16)
        q = jnp.dot(xb, wqb, preferred_element_type=jnp.float32) * SCALE
        qb16 = q.astype(jnp.bfloat16)

        qblk = lax.broadcasted_iota(jnp.int32, (Sq, Skv), 0) // BLK
        kblk = lax.broadcasted_iota(jnp.int32, (Sq, Skv), 1) // BLK
        mask = (qblk == kblk) | (kblk == 0) | ((qblk + kblk) % 3 == 0)

        for h in range(H_PER):
            slot = h % 2
            pending[0].wait()
            pending[1].wait()
            if h + 1 < H_PER:
                pending = kv_copies(h + 1, (h + 1) % 2)
                pending[0].start()
                pending[1].start()
            qh = qb16[:, h * Dh:(h + 1) * Dh]
            kh = kbuf[slot].astype(jnp.bfloat16)
            scores = lax.dot_general(
                qh, kh, (((1,), (1,)), ((), ())),
                preferred_element_type=jnp.float32)
            scores = jnp.where(mask, scores, -1e9)
            m = jnp.max(scores, axis=1, keepdims=True)
            w = jnp.exp(scores - m)
            s = jnp.sum(w, axis=1, keepdims=True)
            vh = vbuf[slot].astype(jnp.bfloat16)
            ctx_h = jnp.dot(w.astype(jnp.bfloat16), vh,
                            preferred_element_type=jnp.float32) / s
            ctx_ref[:, h * Dh:(h + 1) * Dh] = ctx_h.astype(jnp.bfloat16)

        wob = wo_ref[...].astype(jnp.bfloat16)
        partial = jnp.dot(ctx_ref[...], wob,
                          preferred_element_type=jnp.float32)
        pbuf[...] = partial.astype(jnp.bfloat16)

        barrier = pltpu.get_barrier_semaphore()
        p1 = jnp.bitwise_xor(my, 1)
        p2 = 3 - my
        for nbr in (p1, p2):
            pl.semaphore_signal(barrier, inc=1, device_id=(nbr,),
                                device_id_type=pl.DeviceIdType.MESH)
        pl.semaphore_wait(barrier, 2)

        ex1 = pltpu.make_async_remote_copy(
            src_ref=pbuf, dst_ref=rbuf1,
            send_sem=send_sems.at[0], recv_sem=recv_sems.at[0],
            device_id=(p1,), device_id_type=pl.DeviceIdType.MESH)
        ex1.start()
        ex1.wait()
        sum1 = partial + rbuf1[...].astype(jnp.float32)
        sbuf2[...] = sum1.astype(jnp.bfloat16)

        ex2 = pltpu.make_async_remote_copy(
            src_ref=sbuf2, dst_ref=rbuf2,
            send_sem=send_sems.at[1], recv_sem=recv_sems.at[1],
            device_id=(p2,), device_id_type=pl.DeviceIdType.MESH)
        ex2.start()
        ex2.wait()
        out_ref[0] = sum1 + rbuf2[...].astype(jnp.float32)

    return pl.pallas_call(
        body,
        out_shape=jax.ShapeDtypeStruct((B, Sq, Dm), jnp.float32),
        in_specs=[
            pl.BlockSpec(memory_space=pltpu.VMEM),
            pl.BlockSpec(memory_space=pltpu.VMEM),
            pl.BlockSpec(memory_space=pl.ANY),
            pl.BlockSpec(memory_space=pl.ANY),
            pl.BlockSpec(memory_space=pltpu.VMEM),
        ],
        out_specs=pl.BlockSpec(memory_space=pltpu.VMEM),
        scratch_shapes=[
            pltpu.VMEM((2, Skv, Dh), jnp.float32),
            pltpu.VMEM((2, Skv, Dh), jnp.float32),
            pltpu.VMEM((Sq, Dq), jnp.bfloat16),
            pltpu.VMEM((Sq, Dm), jnp.bfloat16),
            pltpu.VMEM((Sq, Dm), jnp.bfloat16),
            pltpu.VMEM((Sq, Dm), jnp.bfloat16),
            pltpu.VMEM((Sq, Dm), jnp.bfloat16),
            pltpu.SemaphoreType.DMA((2, 2)),
            pltpu.SemaphoreType.DMA((2,)),
            pltpu.SemaphoreType.DMA((2,)),
        ],
        compiler_params=pltpu.CompilerParams(collective_id=0),
    )(x, Wq, K_ext, V_ext, Wo)
